# Initial kernel scaffold; baseline (speedup 1.0000x reference)
import jax
import jax.numpy as jnp
from jax.experimental import pallas as pl


def kernel(x_1, x_2, x_3, x_4, edge_index_1, edge_index_2, edge_index_3, edge_index_4, edge_weight_1, edge_weight_2, edge_weight_3, edge_weight_4, Wg1, bg1, gam1, bet1, Wg2, bg2, gam2, bet2, Wfc, bfc, Wf1, bf1, Wf2, bf2):
    raise NotImplementedError("write your pallas kernel here")


# speedup vs baseline: 1.4463x; 1.4463x over previous
import jax
import jax.numpy as jnp
from jax.experimental import pallas as pl
from jax.experimental.pallas import tpu as pltpu

N = 10000
F_IN = 128
H = 512
OUT = 128
NBR = 4
ROWS = 1000
NB = N // ROWS
EPS = 1e-5


def _dot(a, b):
    return jax.lax.dot_general(a, b, (((1,), (0,)), ((), ())),
                               preferred_element_type=jnp.float32)


def _bn_scale_shift(stats, gam, bet):
    mu = stats[0:1] / N
    var = stats[1:2] / N - mu * mu
    a = gam * jax.lax.rsqrt(var + EPS)
    c = bet - mu * a
    return a, c


def _l1_kernel(x_ref, w_ref, b_ref, h_ref, st_ref):
    rb = pl.program_id(1)
    h = _dot(x_ref[0], w_ref[0]) + b_ref[0]
    h_ref[0] = h
    s = jnp.sum(h, axis=0, keepdims=True)
    ss = jnp.sum(h * h, axis=0, keepdims=True)
    st = jnp.concatenate([s, ss], axis=0)

    @pl.when(rb == 0)
    def _():
        st_ref[0] = st

    @pl.when(rb != 0)
    def _():
        st_ref[0] = st_ref[0] + st


def _l2_kernel(h1_ref, st1_ref, gam_ref, bet_ref, w_ref, b_ref,
               h2_ref, st_ref):
    rb = pl.program_id(1)
    a, c = _bn_scale_shift(st1_ref[0], gam_ref[0], bet_ref[0])
    t = jnp.maximum(h1_ref[0] * a + c, 0.0)
    h2 = _dot(t, w_ref[0]) + b_ref[0]
    h2_ref[0] = h2
    s = jnp.sum(h2, axis=0, keepdims=True)
    ss = jnp.sum(h2 * h2, axis=0, keepdims=True)
    st = jnp.concatenate([s, ss], axis=0)

    @pl.when(rb == 0)
    def _():
        st_ref[0] = st

    @pl.when(rb != 0)
    def _():
        st_ref[0] = st_ref[0] + st


def _wc_kernel(wfc_ref, wf1_ref, bfc_ref, wc_ref, bc_ref):
    wc_ref[0] = _dot(wfc_ref[0], wf1_ref[...])
    bc_ref[0] = _dot(bfc_ref[0], wf1_ref[...])


def _l3_kernel(h2_ref, st2_ref, gam_ref, bet_ref, wc_ref, bc_ref,
               bf1_ref, wf2_ref, bf2_ref, out_ref):
    acc = jnp.broadcast_to(bf1_ref[...], (ROWS, H))
    for i in range(NBR):
        a, c = _bn_scale_shift(st2_ref[i], gam_ref[i], bet_ref[i])
        t = jnp.maximum(h2_ref[i] * a + c, 0.0)
        acc = acc + _dot(t, wc_ref[i]) + bc_ref[i]
    pre = jnp.maximum(acc, 0.0)
    out_ref[...] = _dot(pre, wf2_ref[...]) + bf2_ref[...]


def kernel(x_1, x_2, x_3, x_4, edge_index_1, edge_index_2, edge_index_3,
           edge_index_4, edge_weight_1, edge_weight_2, edge_weight_3,
           edge_weight_4, Wg1, bg1, gam1, bet1, Wg2, bg2, gam2, bet2,
           Wfc, bfc, Wf1, bf1, Wf2, bf2):
    xs = jnp.stack([x_1, x_2, x_3, x_4])
    bg1_ = bg1[:, None, :]
    gam1_ = gam1[:, None, :]
    bet1_ = bet1[:, None, :]
    bg2_ = bg2[:, None, :]
    gam2_ = gam2[:, None, :]
    bet2_ = bet2[:, None, :]
    bfc_ = bfc[:, None, :]
    bf1_ = bf1[None, :]
    bf2_ = bf2[None, :]

    h1, st1 = pl.pallas_call(
        _l1_kernel,
        grid=(NBR, NB),
        in_specs=[
            pl.BlockSpec((1, ROWS, F_IN), lambda i, r: (i, r, 0)),
            pl.BlockSpec((1, F_IN, H), lambda i, r: (i, 0, 0)),
            pl.BlockSpec((1, 1, H), lambda i, r: (i, 0, 0)),
        ],
        out_specs=[
            pl.BlockSpec((1, ROWS, H), lambda i, r: (i, r, 0)),
            pl.BlockSpec((1, 2, H), lambda i, r: (i, 0, 0)),
        ],
        out_shape=[
            jax.ShapeDtypeStruct((NBR, N, H), jnp.float32),
            jax.ShapeDtypeStruct((NBR, 2, H), jnp.float32),
        ],
    )(xs, Wg1, bg1_)

    h2, st2 = pl.pallas_call(
        _l2_kernel,
        grid=(NBR, NB),
        in_specs=[
            pl.BlockSpec((1, ROWS, H), lambda i, r: (i, r, 0)),
            pl.BlockSpec((1, 2, H), lambda i, r: (i, 0, 0)),
            pl.BlockSpec((1, 1, H), lambda i, r: (i, 0, 0)),
            pl.BlockSpec((1, 1, H), lambda i, r: (i, 0, 0)),
            pl.BlockSpec((1, H, H), lambda i, r: (i, 0, 0)),
            pl.BlockSpec((1, 1, H), lambda i, r: (i, 0, 0)),
        ],
        out_specs=[
            pl.BlockSpec((1, ROWS, H), lambda i, r: (i, r, 0)),
            pl.BlockSpec((1, 2, H), lambda i, r: (i, 0, 0)),
        ],
        out_shape=[
            jax.ShapeDtypeStruct((NBR, N, H), jnp.float32),
            jax.ShapeDtypeStruct((NBR, 2, H), jnp.float32),
        ],
    )(h1, st1, gam1_, bet1_, Wg2, bg2_)

    wc, bc = pl.pallas_call(
        _wc_kernel,
        grid=(NBR,),
        in_specs=[
            pl.BlockSpec((1, H, H), lambda i: (i, 0, 0)),
            pl.BlockSpec((H, H), lambda i: (i, 0)),
            pl.BlockSpec((1, 1, H), lambda i: (i, 0, 0)),
        ],
        out_specs=[
            pl.BlockSpec((1, H, H), lambda i: (i, 0, 0)),
            pl.BlockSpec((1, 1, H), lambda i: (i, 0, 0)),
        ],
        out_shape=[
            jax.ShapeDtypeStruct((NBR, H, H), jnp.float32),
            jax.ShapeDtypeStruct((NBR, 1, H), jnp.float32),
        ],
    )(Wfc, Wf1, bfc_)

    out = pl.pallas_call(
        _l3_kernel,
        grid=(NB,),
        in_specs=[
            pl.BlockSpec((NBR, ROWS, H), lambda r: (0, r, 0)),
            pl.BlockSpec((NBR, 2, H), lambda r: (0, 0, 0)),
            pl.BlockSpec((NBR, 1, H), lambda r: (0, 0, 0)),
            pl.BlockSpec((NBR, 1, H), lambda r: (0, 0, 0)),
            pl.BlockSpec((NBR, H, H), lambda r: (0, 0, 0)),
            pl.BlockSpec((NBR, 1, H), lambda r: (0, 0, 0)),
            pl.BlockSpec((1, H), lambda r: (0, 0)),
            pl.BlockSpec((H, OUT), lambda r: (0, 0)),
            pl.BlockSpec((1, OUT), lambda r: (0, 0)),
        ],
        out_specs=pl.BlockSpec((ROWS, OUT), lambda r: (r, 0)),
        out_shape=jax.ShapeDtypeStruct((N, OUT), jnp.float32),
    )(h2, st2, gam2_, bet2_, wc, bc, bf1_, Wf2, bf2_)
    return out


# speedup vs baseline: 2.5553x; 1.7668x over previous
import jax
import jax.numpy as jnp
from jax.experimental import pallas as pl
from jax.experimental.pallas import tpu as pltpu

N = 10000
F_IN = 128
H = 512
OUT = 128
NBR = 4
ROWS = 1000
NB = N // ROWS
EPS = 1e-5
BF = jnp.bfloat16


def _dot(a, b):
    return jax.lax.dot_general(a, b, (((1,), (0,)), ((), ())),
                               preferred_element_type=jnp.float32)


def _dott(a, b):
    return jax.lax.dot_general(a, b, (((0,), (0,)), ((), ())),
                               preferred_element_type=jnp.float32)


def _stats_kernel(x1_ref, x2_ref, x3_ref, x4_ref, m_ref, s_ref):
    r = pl.program_id(0)
    ms, ss = [], []
    for xr in (x1_ref, x2_ref, x3_ref, x4_ref):
        xb = xr[...]
        xh = xb.astype(BF)
        ms.append(jnp.sum(xb, axis=0, keepdims=True))
        ss.append(_dott(xh, xh))
    m = jnp.stack(ms)
    s = jnp.stack(ss)

    @pl.when(r == 0)
    def _():
        m_ref[...] = m
        s_ref[...] = s

    @pl.when(r != 0)
    def _():
        m_ref[...] = m_ref[...] + m
        s_ref[...] = s_ref[...] + s


def _branch_kernel(x1_ref, x2_ref, x3_ref, x4_ref, m_ref, s_ref,
                   wg1_ref, bg1_ref, gam1_ref, bet1_ref, wg2_ref, bg2_ref,
                   h2_ref, st2_ref, ac1_ref):
    r = pl.program_id(0)

    @pl.when(r == 0)
    def _():
        acs = []
        for i in range(NBR):
            w1 = wg1_ref[i]
            w1h = w1.astype(BF)
            p = _dot((m_ref[i] * (1.0 / N)), w1)
            sw = _dot((s_ref[i] * (1.0 / N)).astype(BF), w1h)
            e2 = jnp.sum(sw * w1, axis=0, keepdims=True)
            var = e2 - p * p
            a = gam1_ref[i] * jax.lax.rsqrt(var + EPS)
            c = bet1_ref[i] - (p + bg1_ref[i]) * a
            acs.append(jnp.concatenate([a, c], axis=0))
        ac1_ref[...] = jnp.stack(acs)

    sts = []
    for i, xr in enumerate((x1_ref, x2_ref, x3_ref, x4_ref)):
        xh = xr[...].astype(BF)
        h1 = _dot(xh, wg1_ref[i].astype(BF)) + bg1_ref[i]
        t = jnp.maximum(h1 * ac1_ref[i, 0:1] + ac1_ref[i, 1:2], 0.0)
        h2 = _dot(t.astype(BF), wg2_ref[i].astype(BF)) + bg2_ref[i]
        h2_ref[i] = h2.astype(BF)
        s = jnp.sum(h2, axis=0, keepdims=True)
        ss = jnp.sum(h2 * h2, axis=0, keepdims=True)
        sts.append(jnp.concatenate([s, ss], axis=0))
    st = jnp.stack(sts)

    @pl.when(r == 0)
    def _():
        st2_ref[...] = st

    @pl.when(r != 0)
    def _():
        st2_ref[...] = st2_ref[...] + st


def _head_kernel(h2_ref, st2_ref, gam2_ref, bet2_ref, wfc_ref, wf1_ref,
                 bfc_ref, bf1_ref, wf2_ref, bf2_ref, out_ref,
                 wc_ref, bc_ref):
    r = pl.program_id(0)

    @pl.when(r == 0)
    def _():
        bc = jnp.broadcast_to(bf1_ref[...], (1, H)).astype(jnp.float32)
        for i in range(NBR):
            wf1_i = wf1_ref[i * H:(i + 1) * H, :].astype(BF)
            wc_ref[i] = _dot(wfc_ref[i].astype(BF), wf1_i).astype(BF)
            bc = bc + _dot(bfc_ref[i].astype(BF), wf1_i)
        bc_ref[...] = bc

    acc = jnp.broadcast_to(bc_ref[...], (ROWS, H))
    for i in range(NBR):
        s = st2_ref[i]
        mu = s[0:1] * (1.0 / N)
        var = s[1:2] * (1.0 / N) - mu * mu
        a = gam2_ref[i] * jax.lax.rsqrt(var + EPS)
        c = bet2_ref[i] - mu * a
        t = jnp.maximum(h2_ref[i].astype(jnp.float32) * a + c, 0.0)
        acc = acc + _dot(t.astype(BF), wc_ref[i])
    pre = jnp.maximum(acc, 0.0)
    out_ref[...] = _dot(pre.astype(BF), wf2_ref[...].astype(BF)) + bf2_ref[...]


def kernel(x_1, x_2, x_3, x_4, edge_index_1, edge_index_2, edge_index_3,
           edge_index_4, edge_weight_1, edge_weight_2, edge_weight_3,
           edge_weight_4, Wg1, bg1, gam1, bet1, Wg2, bg2, gam2, bet2,
           Wfc, bfc, Wf1, bf1, Wf2, bf2):
    bg1_ = bg1[:, None, :]
    gam1_ = gam1[:, None, :]
    bet1_ = bet1[:, None, :]
    gam2_ = gam2[:, None, :]
    bet2_ = bet2[:, None, :]
    bg2_ = bg2[:, None, :]
    bfc_ = bfc[:, None, :]
    bf1_ = bf1[None, :]
    bf2_ = bf2[None, :]

    xspec = pl.BlockSpec((ROWS, F_IN), lambda r: (r, 0))
    full3 = lambda shape: pl.BlockSpec(shape, lambda r: (0, 0, 0))

    m, s = pl.pallas_call(
        _stats_kernel,
        grid=(NB,),
        in_specs=[xspec, xspec, xspec, xspec],
        out_specs=[full3((NBR, 1, F_IN)), full3((NBR, F_IN, F_IN))],
        out_shape=[
            jax.ShapeDtypeStruct((NBR, 1, F_IN), jnp.float32),
            jax.ShapeDtypeStruct((NBR, F_IN, F_IN), jnp.float32),
        ],
    )(x_1, x_2, x_3, x_4)

    h2, st2 = pl.pallas_call(
        _branch_kernel,
        grid=(NB,),
        in_specs=[
            xspec, xspec, xspec, xspec,
            full3((NBR, 1, F_IN)), full3((NBR, F_IN, F_IN)),
            full3((NBR, F_IN, H)), full3((NBR, 1, H)),
            full3((NBR, 1, H)), full3((NBR, 1, H)),
            full3((NBR, H, H)), full3((NBR, 1, H)),
        ],
        out_specs=[
            pl.BlockSpec((NBR, ROWS, H), lambda r: (0, r, 0)),
            full3((NBR, 2, H)),
        ],
        out_shape=[
            jax.ShapeDtypeStruct((NBR, N, H), BF),
            jax.ShapeDtypeStruct((NBR, 2, H), jnp.float32),
        ],
        scratch_shapes=[pltpu.VMEM((NBR, 2, H), jnp.float32)],
    )(x_1, x_2, x_3, x_4, m, s, Wg1, bg1_, gam1_, bet1_, Wg2, bg2_)

    out = pl.pallas_call(
        _head_kernel,
        grid=(NB,),
        in_specs=[
            pl.BlockSpec((NBR, ROWS, H), lambda r: (0, r, 0)),
            full3((NBR, 2, H)),
            full3((NBR, 1, H)), full3((NBR, 1, H)),
            full3((NBR, H, H)),
            pl.BlockSpec((NBR * H, H), lambda r: (0, 0)),
            full3((NBR, 1, H)),
            pl.BlockSpec((1, H), lambda r: (0, 0)),
            pl.BlockSpec((H, OUT), lambda r: (0, 0)),
            pl.BlockSpec((1, OUT), lambda r: (0, 0)),
        ],
        out_specs=pl.BlockSpec((ROWS, OUT), lambda r: (r, 0)),
        out_shape=jax.ShapeDtypeStruct((N, OUT), jnp.float32),
        scratch_shapes=[
            pltpu.VMEM((NBR, H, H), BF),
            pltpu.VMEM((1, H), jnp.float32),
        ],
    )(h2, st2, gam2_, bet2_, Wfc, Wf1, bfc_, bf1_, Wf2, bf2_)
    return out


# speedup vs baseline: 2.5671x; 1.0046x over previous
import jax
import jax.numpy as jnp
from jax.experimental import pallas as pl
from jax.experimental.pallas import tpu as pltpu

N = 10000
F_IN = 128
H = 512
OUT = 128
NBR = 4
ROWS = 1000
NB = N // ROWS
EPS = 1e-5
BF = jnp.bfloat16


def _dot(a, b):
    return jax.lax.dot_general(a, b, (((1,), (0,)), ((), ())),
                               preferred_element_type=jnp.float32)


def _dott(a, b):
    return jax.lax.dot_general(a, b, (((0,), (0,)), ((), ())),
                               preferred_element_type=jnp.float32)


def _mega_kernel(x1_ref, x2_ref, x3_ref, x4_ref,
                 wg1_ref, gam1_ref, bet1_ref,
                 wg2_ref, gam2_ref, bet2_ref,
                 wfc_ref, wf1_ref, bfc_ref, bf1_ref, wf2_ref, bf2_ref,
                 out_ref,
                 m_scr, s_scr, w1s_scr, c1_scr, h2_scr, st2_scr,
                 wc_scr, bc_scr, ac2_scr):
    r = pl.program_id(0)
    xrefs = (x1_ref, x2_ref, x3_ref, x4_ref)

    @pl.when(r < NB)
    def _():
        ms, ss = [], []
        for xr in xrefs:
            xb = xr[...]
            xh = xb.astype(BF)
            ms.append(jnp.sum(xb, axis=0, keepdims=True))
            ss.append(_dott(xh, xh))
        m = jnp.stack(ms)
        s = jnp.stack(ss)

        @pl.when(r == 0)
        def _():
            m_scr[...] = m
            s_scr[...] = s

        @pl.when(r != 0)
        def _():
            m_scr[...] = m_scr[...] + m
            s_scr[...] = s_scr[...] + s

    @pl.when(r == NB)
    def _():
        for i in range(NBR):
            w1h = wg1_ref[i]
            w1f = w1h.astype(jnp.float32)
            p = _dot(m_scr[i] * (1.0 / N), w1f)
            sw = _dot((s_scr[i] * (1.0 / N)).astype(BF), w1h)
            e2 = jnp.sum(sw * w1f, axis=0, keepdims=True)
            var = e2 - p * p
            a = gam1_ref[i] * jax.lax.rsqrt(var + EPS)
            c1_scr[i] = bet1_ref[i] - p * a
            w1s_scr[i] = (w1f * a).astype(BF)
        st2_scr[...] = jnp.zeros((NBR, 2, H), jnp.float32)

    @pl.when((r >= NB) & (r < 2 * NB))
    def _():
        l = r - NB
        sts = []
        for i, xr in enumerate(xrefs):
            xh = xr[...].astype(BF)
            h1 = _dot(xh, w1s_scr[i])
            t = jnp.maximum(h1 + c1_scr[i], 0.0)
            h2 = _dot(t.astype(BF), wg2_ref[i])
            h2_scr[i, pl.ds(l * ROWS, ROWS), :] = h2.astype(BF)
            su = jnp.sum(h2, axis=0, keepdims=True)
            ss = jnp.sum(h2 * h2, axis=0, keepdims=True)
            sts.append(jnp.concatenate([su, ss], axis=0))
        st2_scr[...] = st2_scr[...] + jnp.stack(sts)

    @pl.when(r == 2 * NB)
    def _():
        bc = jnp.broadcast_to(bf1_ref[...], (1, H)).astype(jnp.float32)
        for i in range(NBR):
            wf1_i = wf1_ref[i * H:(i + 1) * H, :]
            wc_scr[i] = _dot(wfc_ref[i], wf1_i).astype(BF)
            bc = bc + _dot(bfc_ref[i].astype(BF), wf1_i)
            s = st2_scr[i]
            mu = s[0:1] * (1.0 / N)
            var = s[1:2] * (1.0 / N) - mu * mu
            a2 = gam2_ref[i] * jax.lax.rsqrt(var + EPS)
            c2 = bet2_ref[i] - mu * a2
            ac2_scr[i] = jnp.concatenate([a2, c2], axis=0)
        bc_scr[...] = bc

    @pl.when(r >= 2 * NB)
    def _():
        l = r - 2 * NB
        acc = jnp.broadcast_to(bc_scr[...], (ROWS, H))
        for i in range(NBR):
            h2b = h2_scr[i, pl.ds(l * ROWS, ROWS), :].astype(jnp.float32)
            t = jnp.maximum(h2b * ac2_scr[i, 0:1] + ac2_scr[i, 1:2], 0.0)
            acc = acc + _dot(t.astype(BF), wc_scr[i])
        pre = jnp.maximum(acc, 0.0)
        out_ref[...] = _dot(pre.astype(BF), wf2_ref[...]) + bf2_ref[...]


def kernel(x_1, x_2, x_3, x_4, edge_index_1, edge_index_2, edge_index_3,
           edge_index_4, edge_weight_1, edge_weight_2, edge_weight_3,
           edge_weight_4, Wg1, bg1, gam1, bet1, Wg2, bg2, gam2, bet2,
           Wfc, bfc, Wf1, bf1, Wf2, bf2):
    gam1_ = gam1[:, None, :]
    bet1_ = bet1[:, None, :]
    gam2_ = gam2[:, None, :]
    bet2_ = bet2[:, None, :]
    bfc_ = bfc[:, None, :]
    bf1_ = bf1[None, :]
    bf2_ = bf2[None, :]
    Wg1h = Wg1.astype(BF)
    Wg2h = Wg2.astype(BF)
    Wfch = Wfc.astype(BF)
    Wf1h = Wf1.astype(BF)
    Wf2h = Wf2.astype(BF)

    xspec = pl.BlockSpec(
        (ROWS, F_IN),
        lambda r: (jnp.where(r < NB, r,
                             jnp.where(r < 2 * NB, r - NB, NB - 1)), 0))
    full3 = lambda shape: pl.BlockSpec(shape, lambda r: (0, 0, 0))

    out = pl.pallas_call(
        _mega_kernel,
        grid=(3 * NB,),
        in_specs=[
            xspec, xspec, xspec, xspec,
            full3((NBR, F_IN, H)), full3((NBR, 1, H)), full3((NBR, 1, H)),
            full3((NBR, H, H)), full3((NBR, 1, H)), full3((NBR, 1, H)),
            full3((NBR, H, H)),
            pl.BlockSpec((NBR * H, H), lambda r: (0, 0)),
            full3((NBR, 1, H)),
            pl.BlockSpec((1, H), lambda r: (0, 0)),
            pl.BlockSpec((H, OUT), lambda r: (0, 0)),
            pl.BlockSpec((1, OUT), lambda r: (0, 0)),
        ],
        out_specs=pl.BlockSpec(
            (ROWS, OUT),
            lambda r: (jnp.where(r < 2 * NB, 0, r - 2 * NB), 0)),
        out_shape=jax.ShapeDtypeStruct((N, OUT), jnp.float32),
        scratch_shapes=[
            pltpu.VMEM((NBR, 1, F_IN), jnp.float32),
            pltpu.VMEM((NBR, F_IN, F_IN), jnp.float32),
            pltpu.VMEM((NBR, F_IN, H), BF),
            pltpu.VMEM((NBR, 1, H), jnp.float32),
            pltpu.VMEM((NBR, N, H), BF),
            pltpu.VMEM((NBR, 2, H), jnp.float32),
            pltpu.VMEM((NBR, H, H), BF),
            pltpu.VMEM((1, H), jnp.float32),
            pltpu.VMEM((NBR, 2, H), jnp.float32),
        ],
    )(x_1, x_2, x_3, x_4, Wg1h, gam1_, bet1_, Wg2h, gam2_, bet2_,
      Wfch, Wf1h, bfc_, bf1_, Wf2h, bf2_)
    return out
